# baseline (speedup 1.0000x reference)
import functools

import jax
import jax.numpy as jnp
from jax import lax
from jax.experimental import pallas as pl
from jax.experimental.pallas import tpu as pltpu
from jax.experimental.pallas import tpu_sc as plsc

VOCAB_SIZE = 1_000_000
D = 64
B = 16384

_NC = 2
_NS = 16
_L = 16
_NW = _NC * _NS
_BPW = B // _NW
_GROUPS = _BPW // _L


@functools.partial(
    pl.kernel,
    out_type=jax.ShapeDtypeStruct((B,), jnp.float32),
    mesh=plsc.VectorSubcoreMesh(core_axis_name="c", subcore_axis_name="s"),
    scratch_types=[
        pltpu.VMEM((_BPW,), jnp.int32),
        pltpu.VMEM((_BPW,), jnp.int32),
        pltpu.VMEM((_BPW, D), jnp.float32),
        pltpu.VMEM((_BPW, D), jnp.float32),
        pltpu.VMEM((_BPW,), jnp.float32),
        pltpu.SemaphoreType.DMA,
        pltpu.SemaphoreType.DMA,
    ],
    compiler_params=pltpu.CompilerParams(
        needs_layout_passes=False, use_tc_tiling_on_sc=False),
)
def _w2v_kernel(tgt_hbm, ctx_hbm, win_hbm, wout_hbm, out_hbm,
                idx_t, idx_c, rows_t, rows_c, outv, sem_t, sem_c):
    wid = lax.axis_index("s") * _NC + lax.axis_index("c")
    base = wid * _BPW

    pltpu.sync_copy(tgt_hbm.at[pl.ds(base, _BPW)], idx_t)
    pltpu.sync_copy(ctx_hbm.at[pl.ds(base, _BPW)], idx_c)

    cp_t = pltpu.async_copy(win_hbm.at[idx_t], rows_t, sem_t)
    cp_c = pltpu.async_copy(wout_hbm.at[idx_c], rows_c, sem_c)
    cp_t.wait()
    cp_c.wait()

    lane = lax.iota(jnp.int32, _L)

    def group_body(g, carry):
        rvec = g * _L + lane
        acc = jnp.zeros((_L,), jnp.float32)
        for j in range(D):
            cvec = jnp.full((_L,), j, jnp.int32)
            t = plsc.load_gather(rows_t, [rvec, cvec])
            c = plsc.load_gather(rows_c, [rvec, cvec])
            acc = acc + t * c
        outv[pl.ds(g * _L, _L)] = acc
        return carry

    lax.fori_loop(0, _GROUPS, group_body, 0)

    pltpu.sync_copy(outv, out_hbm.at[pl.ds(base, _BPW)])


def kernel(target_word, context_word, W_in, W_out):
    return _w2v_kernel(target_word.astype(jnp.int32),
                       context_word.astype(jnp.int32),
                       W_in, W_out)
